# Initial kernel scaffold; baseline (speedup 1.0000x reference)
import jax
import jax.numpy as jnp
from jax.experimental import pallas as pl


def kernel(w, beta, x, y, particle_id):
    raise NotImplementedError("write your pallas kernel here")


# speedup vs baseline: 6.5333x; 6.5333x over previous
import functools

import jax
import jax.numpy as jnp
from jax import lax
from jax.experimental import pallas as pl
from jax.experimental.pallas import tpu as pltpu
from jax.experimental.pallas import tpu_sc as plsc

N = 50000
NW = 32
CHUNK = 1568
NPAD = NW * CHUNK - N
NID = 1024
L = 16


def _seg_body(beta_hbm, pid_hbm, init_hbm, pmax_hbm, pbg_hbm,
              beta_v, pid_v, acc_v, red_v, bg_v):
    wid = lax.axis_index("s") * 2 + lax.axis_index("c")
    base = wid * CHUNK
    pltpu.sync_copy(beta_hbm.at[pl.ds(base, CHUNK)], beta_v)
    pltpu.sync_copy(pid_hbm.at[pl.ds(base, CHUNK)], pid_v)
    pltpu.sync_copy(init_hbm, acc_v)

    laneoff = lax.iota(jnp.int32, L) * NID

    def body(i, carry):
        s, c = carry
        ids = pid_v[pl.ds(i * L, L)]
        bet = beta_v[pl.ds(i * L, L)]
        is0 = ids == 0
        s = s + jnp.where(is0, bet, 0.0)
        c = c + jnp.where(is0, 1.0, 0.0)
        idx = laneoff + ids
        g = plsc.load_gather(acc_v, [idx])
        plsc.store_scatter(acc_v, [idx], jnp.maximum(g, bet))
        return (s, c)

    zero = jnp.zeros((L,), jnp.float32)
    s, c = lax.fori_loop(0, CHUNK // L, body, (zero, zero))

    def rbody(j, carry):
        m = acc_v[pl.ds(j * L, L)]
        for k in range(1, L):
            m = jnp.maximum(m, acc_v[pl.ds(j * L + k * NID, L)])
        red_v[pl.ds(j * L, L)] = m
        return carry

    lax.fori_loop(0, NID // L, rbody, 0)

    bg_v[pl.ds(0, L)] = s
    bg_v[pl.ds(L, L)] = c
    pltpu.sync_copy(red_v, pmax_hbm.at[wid])
    pltpu.sync_copy(bg_v, pbg_hbm.at[wid])


_phase1 = functools.partial(
    pl.kernel,
    out_type=[
        jax.ShapeDtypeStruct((NW, NID), jnp.float32),
        jax.ShapeDtypeStruct((NW, 2 * L), jnp.float32),
    ],
    mesh=plsc.VectorSubcoreMesh(core_axis_name="c", subcore_axis_name="s"),
    compiler_params=pltpu.CompilerParams(needs_layout_passes=False),
    scratch_types=[
        pltpu.VMEM((CHUNK,), jnp.float32),
        pltpu.VMEM((CHUNK,), jnp.int32),
        pltpu.VMEM((L * NID,), jnp.float32),
        pltpu.VMEM((NID,), jnp.float32),
        pltpu.VMEM((2 * L,), jnp.float32),
    ],
)(_seg_body)


def _merge_body(pm_ref, pbg_ref, b0_ref, out_ref):
    pm = pm_ref[...]
    m = jnp.max(pm, axis=0, keepdims=True)
    gid = lax.broadcasted_iota(jnp.int32, (1, NID), 1)
    pres = (gid >= 1) & (gid < 1000) & (m >= 0.0)
    pcnt = jnp.sum(pres.astype(jnp.float32))
    bvals = jnp.where(m > 0.0, m, b0_ref[...])
    sig_sum = jnp.sum(jnp.where(pres, 1.0 - bvals, 0.0))
    pbg = pbg_ref[...]
    lane = lax.broadcasted_iota(jnp.int32, (NW, 2 * L), 1)
    bg_sum = jnp.sum(jnp.where(lane < L, pbg, 0.0))
    bg_cnt = jnp.sum(jnp.where(lane >= L, pbg, 0.0))
    sig = sig_sum / pcnt
    bg = bg_sum / jnp.maximum(bg_cnt, 1.0)
    res = jnp.where(bg_cnt > 0.0, sig + 0.1 * bg, 0.0)
    out_ref[...] = jnp.broadcast_to(res, (1, 1))


def kernel(w, beta, x, y, particle_id):
    del w, x, y
    beta_p = jnp.concatenate([beta, jnp.zeros((NPAD,), jnp.float32)])
    pid_p = jnp.concatenate(
        [particle_id, jnp.full((NPAD,), NID - 1, jnp.int32)]
    )
    init = jnp.full((L * NID,), -1.0, jnp.float32)
    pmax, pbg = _phase1(beta_p, pid_p, init)
    out = pl.pallas_call(
        _merge_body,
        out_shape=jax.ShapeDtypeStruct((1, 1), jnp.float32),
    )(pmax, pbg, beta[:1].reshape(1, 1))
    return out[0, 0]


# speedup vs baseline: 6.6495x; 1.0178x over previous
import functools

import jax
import jax.numpy as jnp
from jax import lax
from jax.experimental import pallas as pl
from jax.experimental.pallas import tpu as pltpu
from jax.experimental.pallas import tpu_sc as plsc

N = 50000
NWK = 16
CHUNK = 3136
TAIL = N - (NWK - 1) * CHUNK
NV = CHUNK // 16
NV_TAIL = TAIL // 16
NID = 1024
L = 16
COLS = NID // NWK
NST = 5 * L
ROW = NID + 128


def _loss_body(beta_hbm, pid_hbm, init_hbm, out_hbm,
               beta_v, pid_v, acc_v, red_v, seg_v, st2_v, stf_v, out_v,
               sh_all):
    wid = lax.axis_index("s")
    base = wid * CHUNK

    pad_ids = jnp.full((L,), NID - 1, jnp.int32)
    for t in range(NV_TAIL, NV):
        pid_v[pl.ds(t * L, L)] = pad_ids

    @pl.when(wid < NWK - 1)
    def _():
        pltpu.sync_copy(beta_hbm.at[pl.ds(base, CHUNK)], beta_v)
        pltpu.sync_copy(pid_hbm.at[pl.ds(base, CHUNK)], pid_v)

    @pl.when(wid == NWK - 1)
    def _():
        pltpu.sync_copy(beta_hbm.at[pl.ds(base, TAIL)],
                        beta_v.at[pl.ds(0, TAIL)])
        pltpu.sync_copy(pid_hbm.at[pl.ds(base, TAIL)],
                        pid_v.at[pl.ds(0, TAIL)])

    pltpu.sync_copy(init_hbm, acc_v)

    laneoff = lax.iota(jnp.int32, L) * NID

    def body(i, carry):
        s, c = carry
        ids = pid_v[pl.ds(i * L, L)]
        bet = beta_v[pl.ds(i * L, L)]
        is0 = ids == 0
        s = s + jnp.where(is0, bet, 0.0)
        c = c + jnp.where(is0, 1.0, 0.0)
        idx = laneoff + ids
        g = plsc.load_gather(acc_v, [idx])
        plsc.store_scatter(acc_v, [idx], jnp.maximum(g, bet))
        return (s, c)

    zero = jnp.zeros((L,), jnp.float32)
    s, c = lax.fori_loop(0, NV, body, (zero, zero))

    def rbody(j, carry):
        m = acc_v[pl.ds(j * L, L)]
        for k in range(1, L):
            m = jnp.maximum(m, acc_v[pl.ds(j * L + k * NID, L)])
        red_v[pl.ds(j * L, L)] = m
        return carry

    lax.fori_loop(0, NID // L, rbody, 0)

    pltpu.sync_copy(red_v, sh_all.at[wid, pl.ds(0, NID)])
    plsc.subcore_barrier()

    for k in range(NWK):
        pltpu.sync_copy(sh_all.at[k, pl.ds(wid * COLS, COLS)],
                        seg_v.at[pl.ds(k * COLS, COLS)])
    sig_v = jnp.zeros((L,), jnp.float32)
    pc_v = jnp.zeros((L,), jnp.float32)
    z_v = jnp.zeros((L,), jnp.float32)
    for j in range(COLS // L):
        m = seg_v[pl.ds(j * L, L)]
        for k in range(1, NWK):
            m = jnp.maximum(m, seg_v[pl.ds(k * COLS + j * L, L)])
        gid = lax.iota(jnp.int32, L) + (wid * COLS + j * L)
        pres = (gid >= 1) & (gid < 1000) & (m >= 0.0)
        pc_v = pc_v + jnp.where(pres, 1.0, 0.0)
        sig_v = sig_v + jnp.where(pres, 1.0 - jnp.where(m > 0.0, m, 0.0), 0.0)
        z_v = z_v + jnp.where(pres & (m == 0.0), 1.0, 0.0)
    st2_v[pl.ds(0, L)] = sig_v
    st2_v[pl.ds(L, L)] = pc_v
    st2_v[pl.ds(2 * L, L)] = z_v
    st2_v[pl.ds(3 * L, L)] = s
    st2_v[pl.ds(4 * L, L)] = c
    pltpu.sync_copy(st2_v, sh_all.at[wid, pl.ds(NID, NST)])
    plsc.subcore_barrier()

    @pl.when(wid == 0)
    def _():
        for k in range(NWK):
            pltpu.sync_copy(sh_all.at[k, pl.ds(NID, NST)],
                            stf_v.at[pl.ds(k * NST, NST)])
        sig_a = jnp.zeros((L,), jnp.float32)
        pc_a = jnp.zeros((L,), jnp.float32)
        z_a = jnp.zeros((L,), jnp.float32)
        bgs_a = jnp.zeros((L,), jnp.float32)
        bgc_a = jnp.zeros((L,), jnp.float32)
        for k in range(NWK):
            sig_a = sig_a + stf_v[pl.ds(k * NST, L)]
            pc_a = pc_a + stf_v[pl.ds(k * NST + L, L)]
            z_a = z_a + stf_v[pl.ds(k * NST + 2 * L, L)]
            bgs_a = bgs_a + stf_v[pl.ds(k * NST + 3 * L, L)]
            bgc_a = bgc_a + stf_v[pl.ds(k * NST + 4 * L, L)]
        b0 = beta_v[pl.ds(0, L)][0]
        ones = jnp.ones((L,), jnp.float32)
        v_sig = ones * jnp.sum(sig_a) - (ones * jnp.sum(z_a)) * (ones * b0)
        v_pc = ones * jnp.sum(pc_a)
        v_bgs = ones * jnp.sum(bgs_a)
        v_bgc = ones * jnp.sum(bgc_a)
        v_out = v_sig / v_pc + 0.1 * (v_bgs / jnp.maximum(v_bgc, 1.0))
        out_v[...] = jnp.where(v_bgc > 0.0, v_out, 0.0)
        pltpu.sync_copy(out_v, out_hbm)


_loss = functools.partial(
    pl.kernel,
    out_type=jax.ShapeDtypeStruct((L,), jnp.float32),
    mesh=plsc.VectorSubcoreMesh(
        core_axis_name="c", subcore_axis_name="s",
        num_cores=1, num_subcores=NWK,
    ),
    compiler_params=pltpu.CompilerParams(needs_layout_passes=False),
    scratch_types=[
        pltpu.VMEM((CHUNK,), jnp.float32),
        pltpu.VMEM((CHUNK,), jnp.int32),
        pltpu.VMEM((L * NID,), jnp.float32),
        pltpu.VMEM((NID,), jnp.float32),
        pltpu.VMEM((NID,), jnp.float32),
        pltpu.VMEM((NST,), jnp.float32),
        pltpu.VMEM((NWK * NST,), jnp.float32),
        pltpu.VMEM((L,), jnp.float32),
        pltpu.VMEM_SHARED((NWK, ROW), jnp.float32),
    ],
)(_loss_body)


def kernel(w, beta, x, y, particle_id):
    del w, x, y
    init = jnp.full((L * NID,), -1.0, jnp.float32)
    out = _loss(beta, particle_id, init)
    return out[0]


# speedup vs baseline: 7.6542x; 1.1511x over previous
import functools

import jax
import jax.numpy as jnp
from jax import lax
from jax.experimental import pallas as pl
from jax.experimental.pallas import tpu as pltpu
from jax.experimental.pallas import tpu_sc as plsc

N = 50000
NWK = 16
CHUNK = 3136
TAIL = N - (NWK - 1) * CHUNK
NV = CHUNK // 16
NV_TAIL = TAIL // 16
NID = 1024
L = 16
COLS = NID // NWK
NST = 5 * L
NSTP = 128
ROW = NID + NWK * NSTP


def _loss_body(beta_hbm, pid_hbm, init_hbm, out_hbm,
               beta_v, pid_v, acc_v, red_v, seg_v, st2_v, stf_v, out_v,
               sem, sh_all):
    wid = lax.axis_index("s")
    base = wid * CHUNK

    pad_ids = jnp.full((L,), NID - 1, jnp.int32)
    for t in range(NV_TAIL, NV):
        pid_v[pl.ds(t * L, L)] = pad_ids

    cp_init = pltpu.async_copy(init_hbm, acc_v, sem)

    @pl.when(wid < NWK - 1)
    def _():
        pltpu.async_copy(beta_hbm.at[pl.ds(base, CHUNK)], beta_v, sem).wait()
        pltpu.async_copy(pid_hbm.at[pl.ds(base, CHUNK)], pid_v, sem).wait()

    @pl.when(wid == NWK - 1)
    def _():
        pltpu.async_copy(beta_hbm.at[pl.ds(base, TAIL)],
                         beta_v.at[pl.ds(0, TAIL)], sem).wait()
        pltpu.async_copy(pid_hbm.at[pl.ds(base, TAIL)],
                         pid_v.at[pl.ds(0, TAIL)], sem).wait()

    cp_init.wait()

    laneoff = lax.iota(jnp.int32, L) * NID

    def body(i, carry):
        s, c = carry
        for u in range(2):
            ids = pid_v[pl.ds(i * 2 * L + u * L, L)]
            bet = beta_v[pl.ds(i * 2 * L + u * L, L)]
            is0 = ids == 0
            s = s + jnp.where(is0, bet, 0.0)
            c = c + jnp.where(is0, 1.0, 0.0)
            idx = laneoff + ids
            g = plsc.load_gather(acc_v, [idx])
            plsc.store_scatter(acc_v, [idx], jnp.maximum(g, bet))
        return (s, c)

    zero = jnp.zeros((L,), jnp.float32)
    s, c = lax.fori_loop(0, NV // 2, body, (zero, zero))

    def rbody(j, carry):
        m = acc_v[pl.ds(j * L, L)]
        for k in range(1, L):
            m = jnp.maximum(m, acc_v[pl.ds(j * L + k * NID, L)])
        red_v[pl.ds(j * L, L)] = m
        return carry

    lax.fori_loop(0, NID // L, rbody, 0)

    cps = [pltpu.async_copy(red_v.at[pl.ds(cb * COLS, COLS)],
                            sh_all.at[cb, pl.ds(wid * COLS, COLS)], sem)
           for cb in range(NWK)]
    for cp in cps:
        cp.wait()
    plsc.subcore_barrier()

    pltpu.sync_copy(sh_all.at[wid, pl.ds(0, NID)], seg_v)
    sig_v = jnp.zeros((L,), jnp.float32)
    pc_v = jnp.zeros((L,), jnp.float32)
    z_v = jnp.zeros((L,), jnp.float32)
    for j in range(COLS // L):
        m = seg_v[pl.ds(j * L, L)]
        for k in range(1, NWK):
            m = jnp.maximum(m, seg_v[pl.ds(k * COLS + j * L, L)])
        gid = lax.iota(jnp.int32, L) + (wid * COLS + j * L)
        pres = (gid >= 1) & (gid < 1000) & (m >= 0.0)
        pc_v = pc_v + jnp.where(pres, 1.0, 0.0)
        sig_v = sig_v + jnp.where(pres, 1.0 - jnp.where(m > 0.0, m, 0.0), 0.0)
        z_v = z_v + jnp.where(pres & (m == 0.0), 1.0, 0.0)
    st2_v[pl.ds(0, L)] = sig_v
    st2_v[pl.ds(L, L)] = pc_v
    st2_v[pl.ds(2 * L, L)] = z_v
    st2_v[pl.ds(3 * L, L)] = s
    st2_v[pl.ds(4 * L, L)] = c
    row0 = wid * 0
    pltpu.sync_copy(st2_v, sh_all.at[row0, pl.ds(NID + wid * NSTP, NSTP)])
    plsc.subcore_barrier()

    @pl.when(wid == 0)
    def _():
        pltpu.sync_copy(sh_all.at[row0, pl.ds(NID, NWK * NSTP)], stf_v)
        sig_a = jnp.zeros((L,), jnp.float32)
        pc_a = jnp.zeros((L,), jnp.float32)
        z_a = jnp.zeros((L,), jnp.float32)
        bgs_a = jnp.zeros((L,), jnp.float32)
        bgc_a = jnp.zeros((L,), jnp.float32)
        for k in range(NWK):
            sig_a = sig_a + stf_v[pl.ds(k * NSTP, L)]
            pc_a = pc_a + stf_v[pl.ds(k * NSTP + L, L)]
            z_a = z_a + stf_v[pl.ds(k * NSTP + 2 * L, L)]
            bgs_a = bgs_a + stf_v[pl.ds(k * NSTP + 3 * L, L)]
            bgc_a = bgc_a + stf_v[pl.ds(k * NSTP + 4 * L, L)]
        b0 = beta_v[pl.ds(0, L)][0]
        ones = jnp.ones((L,), jnp.float32)
        v_sig = ones * jnp.sum(sig_a) - (ones * jnp.sum(z_a)) * (ones * b0)
        v_pc = ones * jnp.sum(pc_a)
        v_bgs = ones * jnp.sum(bgs_a)
        v_bgc = ones * jnp.sum(bgc_a)
        v_out = v_sig / v_pc + 0.1 * (v_bgs / jnp.maximum(v_bgc, 1.0))
        out_v[...] = jnp.where(v_bgc > 0.0, v_out, 0.0)
        pltpu.sync_copy(out_v, out_hbm)


_loss = functools.partial(
    pl.kernel,
    out_type=jax.ShapeDtypeStruct((L,), jnp.float32),
    mesh=plsc.VectorSubcoreMesh(
        core_axis_name="c", subcore_axis_name="s",
        num_cores=1, num_subcores=NWK,
    ),
    compiler_params=pltpu.CompilerParams(needs_layout_passes=False),
    scratch_types=[
        pltpu.VMEM((CHUNK,), jnp.float32),
        pltpu.VMEM((CHUNK,), jnp.int32),
        pltpu.VMEM((L * NID,), jnp.float32),
        pltpu.VMEM((NID,), jnp.float32),
        pltpu.VMEM((NID,), jnp.float32),
        pltpu.VMEM((NSTP,), jnp.float32),
        pltpu.VMEM((NWK * NSTP,), jnp.float32),
        pltpu.VMEM((L,), jnp.float32),
        pltpu.SemaphoreType.DMA,
        pltpu.VMEM_SHARED((NWK, ROW), jnp.float32),
    ],
)(_loss_body)


def kernel(w, beta, x, y, particle_id):
    del w, x, y
    init = jnp.full((L * NID,), -1.0, jnp.float32)
    out = _loss(beta, particle_id, init)
    return out[0]
